# baseline (speedup 1.0000x reference)
import dataclasses

import jax
import jax.numpy as jnp
from jax import lax
from jax.experimental import pallas as pl
from jax.experimental.pallas import tpu as pltpu
from jax.experimental.pallas import tpu_sc as plsc

B, N, T, C, H, W = 2, 8, 4, 3, 256, 256
P = N * H * W
NR = B * T
HWD = H * W
NS = 16
PPT = P // NS
CH = 2048
NCH = PPT // CH
ZP = 66048
STRIP = ZP // NS
GRP = 2


def _proj_body(params_ref, depth_ref, pix_ref, zc_ref):
    g = pl.program_id(0)
    d = depth_ref[0]
    u = lax.broadcasted_iota(jnp.int32, (H, W), 1).astype(jnp.float32)
    v = lax.broadcasted_iota(jnp.int32, (H, W), 0).astype(jnp.float32)

    def s(j):
        return params_ref[g, j]

    def bf(t):
        return t.astype(jnp.bfloat16).astype(jnp.float32)

    x = (u - s(0)) / s(2) * d
    y = (v - s(1)) / s(3) * d
    z = d
    xb, yb, zb = bf(x - s(13)), bf(y - s(14)), bf(z - s(15))
    pwx = xb * bf(s(4)) + yb * bf(s(7)) + zb * bf(s(10))
    pwy = xb * bf(s(5)) + yb * bf(s(8)) + zb * bf(s(11))
    pwz = xb * bf(s(6)) + yb * bf(s(9)) + zb * bf(s(12))
    wxb, wyb, wzb = bf(pwx), bf(pwy), bf(pwz)
    X = wxb * bf(s(16)) + wyb * bf(s(17)) + wzb * bf(s(18)) + s(25)
    Y = wxb * bf(s(19)) + wyb * bf(s(20)) + wzb * bf(s(21)) + s(26)
    Z = wxb * bf(s(22)) + wyb * bf(s(23)) + wzb * bf(s(24)) + s(27)
    px = bf(s(28)) * bf(X) + bf(s(30)) * bf(Z)
    py = bf(s(29)) * bf(Y) + bf(s(31)) * bf(Z)
    zs = jnp.maximum(Z, 1e-6)
    ui = jnp.floor(px / zs).astype(jnp.int32)
    vi = jnp.floor(py / zs).astype(jnp.int32)
    valid = (Z > 1e-6) & (ui >= 0) & (ui < W) & (vi >= 0) & (vi < H)
    pix_ref[0] = jnp.where(valid, vi * W + ui, HWD)
    zc_ref[0] = jnp.where(valid, Z, jnp.float32(1e9))


def _project(depths_fl, params):
    return pl.pallas_call(
        _proj_body,
        grid=(B * T * N,),
        in_specs=[
            pl.BlockSpec(memory_space=pltpu.SMEM),
            pl.BlockSpec((1, H, W), lambda g: (g // (T * N) * N + g % N, 0, 0)),
        ],
        out_specs=[
            pl.BlockSpec((1, H, W), lambda g: (g, 0, 0)),
            pl.BlockSpec((1, H, W), lambda g: (g, 0, 0)),
        ],
        out_shape=[
            jax.ShapeDtypeStruct((B * T * N, H, W), jnp.int32),
            jax.ShapeDtypeStruct((B * T * N, H, W), jnp.float32),
        ],
    )(params, depths_fl)


def _take16(a, i):
    dn = lax.GatherDimensionNumbers(
        offset_dims=(), collapsed_slice_dims=(0,), start_index_map=(0,))
    return lax.gather(a, i[:, None], dn, (1,),
                      mode=lax.GatherScatterMode.PROMISE_IN_BOUNDS)


def _segmin_scatter(zpart, kv, zv):
    idx = lax.iota(jnp.int32, 16)
    k, z = plsc.sort_key_val(kv, zv)
    for d in (1, 2, 4, 8):
        sh = jnp.maximum(idx - d, 0)
        kd = _take16(k, sh)
        zd = _take16(z, sh)
        z = jnp.where((kd == k) & (idx >= d), jnp.minimum(z, zd), z)
    kn = _take16(k, jnp.minimum(idx + 1, 15))
    last = (idx == 15) | (kn != k)
    cur = plsc.load_gather(zpart, [k])
    plsc.store_scatter(zpart, [k], jnp.minimum(z, cur), mask=last)


def _sc_render(pix_hbm, zc_hbm, cols_hbm, outr_hbm, outg_hbm, outb_hbm,
               zpart, pixb, zcb, colr, colg, colb,
               pixb2, zcb2, colr2, colg2, colb2, str_, std,
               mstrip, tbuf, tb1, zslots,
               accr, accg, accb, accd, sem, sema, semb):
    cid = lax.axis_index("c")
    sid = lax.axis_index("s")

    @pl.loop(0, T)
    def _render(rl):
        r = cid * T + rl
        b = cid

        def issue_a(c, pb, zb, s):
            base = sid * PPT + c * CH
            pltpu.async_copy(pix_hbm.at[pl.ds(r * P + base, CH)], pb, s)
            pltpu.async_copy(zc_hbm.at[pl.ds(r * P + base, CH)], zb, s)

        def drain_a(c, pb, zb, s):
            base = sid * PPT + c * CH
            pltpu.make_async_copy(pix_hbm.at[pl.ds(r * P + base, CH)],
                                  pb, s).wait()
            pltpu.make_async_copy(zc_hbm.at[pl.ds(r * P + base, CH)],
                                  zb, s).wait()

        def proc_a(pb, zb):
            @pl.loop(0, CH, step=16, unroll=2)
            def _vec(i):
                _segmin_scatter(zpart, pb[pl.ds(i, 16)], zb[pl.ds(i, 16)])

        issue_a(0, pixb, zcb, sema)

        @pl.loop(0, ZP, step=16)
        def _init(i):
            zpart[pl.ds(i, 16)] = jnp.full((16,), 1e9, jnp.float32)

        @pl.loop(0, NCH, step=2)
        def _chunk_a(c):
            issue_a(c + 1, pixb2, zcb2, semb)
            drain_a(c, pixb, zcb, sema)
            proc_a(pixb, zcb)

            @pl.when(c + 2 < NCH)
            def _pf():
                issue_a(c + 2, pixb, zcb, sema)

            drain_a(c + 1, pixb2, zcb2, semb)
            proc_a(pixb2, zcb2)

        so = sid * STRIP

        @pl.loop(0, STRIP, step=16)
        def _minit(i):
            mstrip[pl.ds(i, 16)] = jnp.full((16,), 1e9, jnp.float32)

        @pl.loop(0, NS // GRP)
        def _round(g):
            @pl.when(sid // GRP == g)
            def _pub():
                pltpu.sync_copy(zpart, zslots.at[pl.ds((sid % GRP) * ZP, ZP)])

            plsc.subcore_barrier()

            hs = [pltpu.async_copy(zslots.at[pl.ds(j * ZP + so, STRIP)], tb,
                                   sem)
                  for j, tb in enumerate((tbuf, tb1))]
            for h in hs:
                h.wait()

            @pl.loop(0, STRIP, step=16)
            def _mvec(i):
                sl = pl.ds(i, 16)
                mstrip[sl] = jnp.minimum(
                    mstrip[sl], jnp.minimum(tbuf[sl], tb1[sl]))

            plsc.subcore_barrier()

        pltpu.sync_copy(mstrip, zslots.at[pl.ds(so, STRIP)])
        plsc.subcore_barrier()
        pltpu.sync_copy(zslots.at[pl.ds(0, ZP)], zpart)

        @pl.loop(0, CH, step=16)
        def _zero(i):
            str_[pl.ds(i, 16)] = jnp.zeros((16,), jnp.float32)

        hz = []
        for acc in (accr, accg, accb, accd):
            hz.append(pltpu.async_copy(str_.at[pl.ds(0, CH)],
                                       acc.at[pl.ds(so, CH)], sem))
            hz.append(pltpu.async_copy(str_.at[pl.ds(0, CH)],
                                       acc.at[pl.ds(so + CH, CH)], sem))
            hz.append(pltpu.async_copy(str_.at[pl.ds(0, 32)],
                                       acc.at[pl.ds(so + 2 * CH, 32)], sem))
        for h in hz:
            h.wait()
        plsc.subcore_barrier()

        def srcs_b(c):
            base = sid * PPT + c * CH
            n = base // (H * W)
            off = base % (H * W)
            cbase = ((b * N + n) * C) * (H * W) + off
            return (pix_hbm.at[pl.ds(r * P + base, CH)],
                    zc_hbm.at[pl.ds(r * P + base, CH)],
                    cols_hbm.at[pl.ds(cbase, CH)],
                    cols_hbm.at[pl.ds(cbase + H * W, CH)],
                    cols_hbm.at[pl.ds(cbase + 2 * H * W, CH)])

        def issue_b(c, bufs, s):
            for sr, dsts in zip(srcs_b(c), bufs):
                pltpu.async_copy(sr, dsts, s)

        def drain_b(c, bufs, s):
            for sr, dsts in zip(srcs_b(c), bufs):
                pltpu.make_async_copy(sr, dsts, s).wait()

        def proc_b(pb, zb, cr_, cg_, cb_, sd):
            @pl.loop(0, CH, step=16, unroll=2)
            def _vec(i):
                sl = pl.ds(i, 16)
                k = pb[sl]
                z = zb[sl]
                zv = plsc.load_gather(zpart, [k])
                wf = jnp.where(z <= zv * jnp.float32(1.0 + 1e-4),
                               jnp.float32(1.0), jnp.float32(0.0))
                cr_[sl] = cr_[sl] * wf
                cg_[sl] = cg_[sl] * wf
                cb_[sl] = cb_[sl] * wf
                sd[sl] = wf

            ha = [pltpu.async_copy(cr_, accr.at[pb], sem, add=True),
                  pltpu.async_copy(cg_, accg.at[pb], sem, add=True),
                  pltpu.async_copy(cb_, accb.at[pb], sem, add=True),
                  pltpu.async_copy(sd, accd.at[pb], sem, add=True)]
            for h in ha:
                h.wait()

        bufs0 = (pixb, zcb, colr, colg, colb)
        bufs1 = (pixb2, zcb2, colr2, colg2, colb2)
        issue_b(0, bufs0, sema)

        @pl.loop(0, NCH, step=2)
        def _chunk_b(c):
            issue_b(c + 1, bufs1, semb)
            drain_b(c, bufs0, sema)
            proc_b(*bufs0, str_)

            @pl.when(c + 2 < NCH)
            def _pf():
                issue_b(c + 2, bufs0, sema)

            drain_b(c + 1, bufs1, semb)
            proc_b(*bufs1, std)

        plsc.subcore_barrier()

        zhead = zpart.at[pl.ds(0, STRIP)]
        he = [pltpu.async_copy(accd.at[pl.ds(so, STRIP)], tbuf, sem),
              pltpu.async_copy(accr.at[pl.ds(so, STRIP)], tb1, sem),
              pltpu.async_copy(accg.at[pl.ds(so, STRIP)], mstrip, sem),
              pltpu.async_copy(accb.at[pl.ds(so, STRIP)], zhead, sem)]
        for h in he:
            h.wait()

        @pl.loop(0, STRIP, step=16)
        def _norm(i):
            sl = pl.ds(i, 16)
            dq = jnp.maximum(tbuf[sl], jnp.float32(1.0))
            tb1[sl] = tb1[sl] / dq
            mstrip[sl] = mstrip[sl] / dq
            zpart[sl] = zpart[sl] / dq

        ho = [pltpu.async_copy(tb1, outr_hbm.at[pl.ds(r * ZP + so, STRIP)],
                               sem),
              pltpu.async_copy(mstrip, outg_hbm.at[pl.ds(r * ZP + so, STRIP)],
                               sem),
              pltpu.async_copy(zhead, outb_hbm.at[pl.ds(r * ZP + so, STRIP)],
                               sem)]
        for h in ho:
            h.wait()
        plsc.subcore_barrier()


def _sc_rasterize(pix2, zc2, colsf):
    cp = pltpu.CompilerParams()
    if "needs_layout_passes" in pltpu.CompilerParams.__dataclass_fields__:
        cp = dataclasses.replace(cp, needs_layout_passes=False)
    f32 = jnp.float32
    kern = pl.kernel(
        _sc_render,
        out_type=[jax.ShapeDtypeStruct((NR * ZP,), f32)] * 3,
        mesh=plsc.VectorSubcoreMesh(core_axis_name="c", subcore_axis_name="s"),
        scratch_types=[
            pltpu.VMEM((ZP,), f32),
            pltpu.VMEM((CH,), jnp.int32),
            pltpu.VMEM((CH,), f32),
            pltpu.VMEM((CH,), f32),
            pltpu.VMEM((CH,), f32),
            pltpu.VMEM((CH,), f32),
            pltpu.VMEM((CH,), jnp.int32),
            pltpu.VMEM((CH,), f32),
            pltpu.VMEM((CH,), f32),
            pltpu.VMEM((CH,), f32),
            pltpu.VMEM((CH,), f32),
            pltpu.VMEM((CH,), f32),
            pltpu.VMEM((CH,), f32),
            pltpu.VMEM((STRIP,), f32),
            pltpu.VMEM((STRIP,), f32),
            pltpu.VMEM((STRIP,), f32),
            pltpu.VMEM_SHARED((GRP * ZP,), f32),
            pltpu.VMEM_SHARED((ZP,), f32),
            pltpu.VMEM_SHARED((ZP,), f32),
            pltpu.VMEM_SHARED((ZP,), f32),
            pltpu.VMEM_SHARED((ZP,), f32),
            pltpu.SemaphoreType.DMA,
            pltpu.SemaphoreType.DMA,
            pltpu.SemaphoreType.DMA,
        ],
        compiler_params=cp,
    )
    return kern(pix2, zc2, colsf)


def _frontend(images, depths, extrinsics, intrinsics,
              target_extrinsics, target_intrinsics):
    f32 = jnp.float32
    Rn = extrinsics[:, :, :3, :3]
    tn = extrinsics[:, :, :3, 3]
    Rt = target_extrinsics[:, :, :3, :3]
    tt = target_extrinsics[:, :, :3, 3]

    def bn(a):
        return jnp.broadcast_to(a[:, None, :], (B, T, N))[..., None]

    def bt(a):
        return jnp.broadcast_to(a[:, :, None], (B, T, N))[..., None]

    RnB = jnp.broadcast_to(Rn[:, None], (B, T, N, 3, 3)).reshape(B, T, N, 9)
    RtB = jnp.broadcast_to(Rt[:, :, None], (B, T, N, 3, 3)).reshape(B, T, N, 9)
    tnB = jnp.broadcast_to(tn[:, None], (B, T, N, 3))
    ttB = jnp.broadcast_to(tt[:, :, None], (B, T, N, 3))
    params = jnp.concatenate([
        bn(intrinsics[:, :, 0, 2]), bn(intrinsics[:, :, 1, 2]),
        bn(intrinsics[:, :, 0, 0]), bn(intrinsics[:, :, 1, 1]),
        RnB,
        tnB,
        RtB,
        ttB,
        bt(target_intrinsics[:, :, 0, 0]), bt(target_intrinsics[:, :, 1, 1]),
        bt(target_intrinsics[:, :, 0, 2]), bt(target_intrinsics[:, :, 1, 2]),
    ], axis=-1).astype(f32).reshape(B * T * N, 32)

    depths_fl = depths.reshape(B * N, H, W)
    pix, zc = _project(depths_fl, params)
    pix2 = pix.reshape(NR * P)
    zc2 = zc.reshape(NR * P)
    colsf = images.reshape(B * N * C * H * W)
    return pix2, zc2, colsf


def kernel(images, depths, extrinsics, intrinsics,
           target_extrinsics, target_intrinsics):
    pix2, zc2, colsf = _frontend(images, depths, extrinsics, intrinsics,
                                 target_extrinsics, target_intrinsics)
    outr, outg, outb = (o.reshape(NR, ZP)[:, :HWD]
                        for o in _sc_rasterize(pix2, zc2, colsf))
    img = jnp.stack([outr, outg, outb], axis=-1)
    return img.reshape(B, T, H, W, C)


# speedup vs baseline: 1.0214x; 1.0214x over previous
import dataclasses

import jax
import jax.numpy as jnp
from jax import lax
from jax.experimental import pallas as pl
from jax.experimental.pallas import tpu as pltpu
from jax.experimental.pallas import tpu_sc as plsc

B, N, T, C, H, W = 2, 8, 4, 3, 256, 256
P = N * H * W
NR = B * T
HWD = H * W
NS = 16
PPT = P // NS
CH = 2048
NCH = PPT // CH
ZP = 66048
STRIP = ZP // NS
GRP = 2


def _proj_body(params_ref, depth_ref, pix_ref, zc_ref):
    g = pl.program_id(0)
    d = depth_ref[0]
    u = lax.broadcasted_iota(jnp.int32, (H, W), 1).astype(jnp.float32)
    v = lax.broadcasted_iota(jnp.int32, (H, W), 0).astype(jnp.float32)

    def s(j):
        return params_ref[g, j]

    def bf(t):
        return t.astype(jnp.bfloat16).astype(jnp.float32)

    x = (u - s(0)) / s(2) * d
    y = (v - s(1)) / s(3) * d
    z = d
    xb, yb, zb = bf(x - s(13)), bf(y - s(14)), bf(z - s(15))
    pwx = xb * bf(s(4)) + yb * bf(s(7)) + zb * bf(s(10))
    pwy = xb * bf(s(5)) + yb * bf(s(8)) + zb * bf(s(11))
    pwz = xb * bf(s(6)) + yb * bf(s(9)) + zb * bf(s(12))
    wxb, wyb, wzb = bf(pwx), bf(pwy), bf(pwz)
    X = wxb * bf(s(16)) + wyb * bf(s(17)) + wzb * bf(s(18)) + s(25)
    Y = wxb * bf(s(19)) + wyb * bf(s(20)) + wzb * bf(s(21)) + s(26)
    Z = wxb * bf(s(22)) + wyb * bf(s(23)) + wzb * bf(s(24)) + s(27)
    px = bf(s(28)) * bf(X) + bf(s(30)) * bf(Z)
    py = bf(s(29)) * bf(Y) + bf(s(31)) * bf(Z)
    zs = jnp.maximum(Z, 1e-6)
    ui = jnp.floor(px / zs).astype(jnp.int32)
    vi = jnp.floor(py / zs).astype(jnp.int32)
    valid = (Z > 1e-6) & (ui >= 0) & (ui < W) & (vi >= 0) & (vi < H)
    pix_ref[0] = jnp.where(valid, vi * W + ui, HWD)
    zc_ref[0] = jnp.where(valid, Z, jnp.float32(1e9))


def _project(depths_fl, params):
    return pl.pallas_call(
        _proj_body,
        grid=(B * T * N,),
        in_specs=[
            pl.BlockSpec(memory_space=pltpu.SMEM),
            pl.BlockSpec((1, H, W), lambda g: (g // (T * N) * N + g % N, 0, 0)),
        ],
        out_specs=[
            pl.BlockSpec((1, H, W), lambda g: (g, 0, 0)),
            pl.BlockSpec((1, H, W), lambda g: (g, 0, 0)),
        ],
        out_shape=[
            jax.ShapeDtypeStruct((B * T * N, H, W), jnp.int32),
            jax.ShapeDtypeStruct((B * T * N, H, W), jnp.float32),
        ],
    )(params, depths_fl)


def _take16(a, i):
    dn = lax.GatherDimensionNumbers(
        offset_dims=(), collapsed_slice_dims=(0,), start_index_map=(0,))
    return lax.gather(a, i[:, None], dn, (1,),
                      mode=lax.GatherScatterMode.PROMISE_IN_BOUNDS)


def _segmin_scatter(zpart, kv, zv):
    idx = lax.iota(jnp.int32, 16)
    k, z = plsc.sort_key_val(kv, zv)
    for d in (1, 2, 4, 8):
        sh = jnp.maximum(idx - d, 0)
        kd = _take16(k, sh)
        zd = _take16(z, sh)
        z = jnp.where((kd == k) & (idx >= d), jnp.minimum(z, zd), z)
    kn = _take16(k, jnp.minimum(idx + 1, 15))
    last = (idx == 15) | (kn != k)
    cur = plsc.load_gather(zpart, [k])
    plsc.store_scatter(zpart, [k], jnp.minimum(z, cur), mask=last)


def _sc_render(pix_hbm, zc_hbm, cols_hbm, outr_hbm, outg_hbm, outb_hbm,
               zpart, pixb, zcb, colr, colg, colb,
               pixb2, zcb2, colr2, colg2, colb2, str_, std,
               mstrip, tbuf, tb1, zslots,
               accr, accg, accb, accd, sem, sema, semb):
    cid = lax.axis_index("c")
    sid = lax.axis_index("s")

    @pl.loop(0, T)
    def _render(rl):
        r = cid * T + rl
        b = cid

        def issue_a(c, pb, zb, s):
            base = sid * PPT + c * CH
            pltpu.async_copy(pix_hbm.at[pl.ds(r * P + base, CH)], pb, s)
            pltpu.async_copy(zc_hbm.at[pl.ds(r * P + base, CH)], zb, s)

        def drain_a(c, pb, zb, s):
            base = sid * PPT + c * CH
            pltpu.make_async_copy(pix_hbm.at[pl.ds(r * P + base, CH)],
                                  pb, s).wait()
            pltpu.make_async_copy(zc_hbm.at[pl.ds(r * P + base, CH)],
                                  zb, s).wait()

        def proc_a(pb, zb):
            @pl.loop(0, CH, step=16)
            def _vec(i):
                _segmin_scatter(zpart, pb[pl.ds(i, 16)], zb[pl.ds(i, 16)])

        issue_a(0, pixb, zcb, sema)

        @pl.loop(0, ZP, step=16)
        def _init(i):
            zpart[pl.ds(i, 16)] = jnp.full((16,), 1e9, jnp.float32)

        @pl.loop(0, NCH, step=2)
        def _chunk_a(c):
            issue_a(c + 1, pixb2, zcb2, semb)
            drain_a(c, pixb, zcb, sema)
            proc_a(pixb, zcb)

            @pl.when(c + 2 < NCH)
            def _pf():
                issue_a(c + 2, pixb, zcb, sema)

            drain_a(c + 1, pixb2, zcb2, semb)
            proc_a(pixb2, zcb2)

        so = sid * STRIP

        @pl.loop(0, STRIP, step=16)
        def _minit(i):
            mstrip[pl.ds(i, 16)] = jnp.full((16,), 1e9, jnp.float32)

        @pl.loop(0, NS // GRP)
        def _round(g):
            @pl.when(sid // GRP == g)
            def _pub():
                pltpu.sync_copy(zpart, zslots.at[pl.ds((sid % GRP) * ZP, ZP)])

            plsc.subcore_barrier()

            hs = [pltpu.async_copy(zslots.at[pl.ds(j * ZP + so, STRIP)], tb,
                                   sem)
                  for j, tb in enumerate((tbuf, tb1))]
            for h in hs:
                h.wait()

            @pl.loop(0, STRIP, step=16)
            def _mvec(i):
                sl = pl.ds(i, 16)
                mstrip[sl] = jnp.minimum(
                    mstrip[sl], jnp.minimum(tbuf[sl], tb1[sl]))

            plsc.subcore_barrier()

        pltpu.sync_copy(mstrip, zslots.at[pl.ds(so, STRIP)])
        plsc.subcore_barrier()
        pltpu.sync_copy(zslots.at[pl.ds(0, ZP)], zpart)

        @pl.loop(0, CH, step=16)
        def _zero(i):
            str_[pl.ds(i, 16)] = jnp.zeros((16,), jnp.float32)

        hz = []
        for acc in (accr, accg, accb, accd):
            hz.append(pltpu.async_copy(str_.at[pl.ds(0, CH)],
                                       acc.at[pl.ds(so, CH)], sem))
            hz.append(pltpu.async_copy(str_.at[pl.ds(0, CH)],
                                       acc.at[pl.ds(so + CH, CH)], sem))
            hz.append(pltpu.async_copy(str_.at[pl.ds(0, 32)],
                                       acc.at[pl.ds(so + 2 * CH, 32)], sem))
        for h in hz:
            h.wait()
        plsc.subcore_barrier()

        def srcs_b(c):
            base = sid * PPT + c * CH
            n = base // (H * W)
            off = base % (H * W)
            cbase = ((b * N + n) * C) * (H * W) + off
            return (pix_hbm.at[pl.ds(r * P + base, CH)],
                    zc_hbm.at[pl.ds(r * P + base, CH)],
                    cols_hbm.at[pl.ds(cbase, CH)],
                    cols_hbm.at[pl.ds(cbase + H * W, CH)],
                    cols_hbm.at[pl.ds(cbase + 2 * H * W, CH)])

        def issue_b(c, bufs, s):
            for sr, dsts in zip(srcs_b(c), bufs):
                pltpu.async_copy(sr, dsts, s)

        def drain_b(c, bufs, s):
            for sr, dsts in zip(srcs_b(c), bufs):
                pltpu.make_async_copy(sr, dsts, s).wait()

        def proc_b(pb, zb, cr_, cg_, cb_, sd):
            @pl.loop(0, CH, step=16)
            def _vec(i):
                sl = pl.ds(i, 16)
                k = pb[sl]
                z = zb[sl]
                zv = plsc.load_gather(zpart, [k])
                wf = jnp.where(z <= zv * jnp.float32(1.0 + 1e-4),
                               jnp.float32(1.0), jnp.float32(0.0))
                cr_[sl] = cr_[sl] * wf
                cg_[sl] = cg_[sl] * wf
                cb_[sl] = cb_[sl] * wf
                sd[sl] = wf

            ha = [pltpu.async_copy(cr_, accr.at[pb], sem, add=True),
                  pltpu.async_copy(cg_, accg.at[pb], sem, add=True),
                  pltpu.async_copy(cb_, accb.at[pb], sem, add=True),
                  pltpu.async_copy(sd, accd.at[pb], sem, add=True)]
            for h in ha:
                h.wait()

        bufs0 = (pixb, zcb, colr, colg, colb)
        bufs1 = (pixb2, zcb2, colr2, colg2, colb2)
        issue_b(0, bufs0, sema)

        @pl.loop(0, NCH, step=2)
        def _chunk_b(c):
            issue_b(c + 1, bufs1, semb)
            drain_b(c, bufs0, sema)
            proc_b(*bufs0, str_)

            @pl.when(c + 2 < NCH)
            def _pf():
                issue_b(c + 2, bufs0, sema)

            drain_b(c + 1, bufs1, semb)
            proc_b(*bufs1, std)

        plsc.subcore_barrier()

        zhead = zpart.at[pl.ds(0, STRIP)]
        he = [pltpu.async_copy(accd.at[pl.ds(so, STRIP)], tbuf, sem),
              pltpu.async_copy(accr.at[pl.ds(so, STRIP)], tb1, sem),
              pltpu.async_copy(accg.at[pl.ds(so, STRIP)], mstrip, sem),
              pltpu.async_copy(accb.at[pl.ds(so, STRIP)], zhead, sem)]
        for h in he:
            h.wait()

        @pl.loop(0, STRIP, step=16)
        def _norm(i):
            sl = pl.ds(i, 16)
            dq = jnp.maximum(tbuf[sl], jnp.float32(1.0))
            tb1[sl] = tb1[sl] / dq
            mstrip[sl] = mstrip[sl] / dq
            zpart[sl] = zpart[sl] / dq

        ho = [pltpu.async_copy(tb1, outr_hbm.at[pl.ds(r * ZP + so, STRIP)],
                               sem),
              pltpu.async_copy(mstrip, outg_hbm.at[pl.ds(r * ZP + so, STRIP)],
                               sem),
              pltpu.async_copy(zhead, outb_hbm.at[pl.ds(r * ZP + so, STRIP)],
                               sem)]
        for h in ho:
            h.wait()
        plsc.subcore_barrier()


def _sc_rasterize(pix2, zc2, colsf):
    cp = pltpu.CompilerParams()
    if "needs_layout_passes" in pltpu.CompilerParams.__dataclass_fields__:
        cp = dataclasses.replace(cp, needs_layout_passes=False)
    f32 = jnp.float32
    kern = pl.kernel(
        _sc_render,
        out_type=[jax.ShapeDtypeStruct((NR * ZP,), f32)] * 3,
        mesh=plsc.VectorSubcoreMesh(core_axis_name="c", subcore_axis_name="s"),
        scratch_types=[
            pltpu.VMEM((ZP,), f32),
            pltpu.VMEM((CH,), jnp.int32),
            pltpu.VMEM((CH,), f32),
            pltpu.VMEM((CH,), f32),
            pltpu.VMEM((CH,), f32),
            pltpu.VMEM((CH,), f32),
            pltpu.VMEM((CH,), jnp.int32),
            pltpu.VMEM((CH,), f32),
            pltpu.VMEM((CH,), f32),
            pltpu.VMEM((CH,), f32),
            pltpu.VMEM((CH,), f32),
            pltpu.VMEM((CH,), f32),
            pltpu.VMEM((CH,), f32),
            pltpu.VMEM((STRIP,), f32),
            pltpu.VMEM((STRIP,), f32),
            pltpu.VMEM((STRIP,), f32),
            pltpu.VMEM_SHARED((GRP * ZP,), f32),
            pltpu.VMEM_SHARED((ZP,), f32),
            pltpu.VMEM_SHARED((ZP,), f32),
            pltpu.VMEM_SHARED((ZP,), f32),
            pltpu.VMEM_SHARED((ZP,), f32),
            pltpu.SemaphoreType.DMA,
            pltpu.SemaphoreType.DMA,
            pltpu.SemaphoreType.DMA,
        ],
        compiler_params=cp,
    )
    return kern(pix2, zc2, colsf)


def _frontend(images, depths, extrinsics, intrinsics,
              target_extrinsics, target_intrinsics):
    f32 = jnp.float32
    Rn = extrinsics[:, :, :3, :3]
    tn = extrinsics[:, :, :3, 3]
    Rt = target_extrinsics[:, :, :3, :3]
    tt = target_extrinsics[:, :, :3, 3]

    def bn(a):
        return jnp.broadcast_to(a[:, None, :], (B, T, N))[..., None]

    def bt(a):
        return jnp.broadcast_to(a[:, :, None], (B, T, N))[..., None]

    RnB = jnp.broadcast_to(Rn[:, None], (B, T, N, 3, 3)).reshape(B, T, N, 9)
    RtB = jnp.broadcast_to(Rt[:, :, None], (B, T, N, 3, 3)).reshape(B, T, N, 9)
    tnB = jnp.broadcast_to(tn[:, None], (B, T, N, 3))
    ttB = jnp.broadcast_to(tt[:, :, None], (B, T, N, 3))
    params = jnp.concatenate([
        bn(intrinsics[:, :, 0, 2]), bn(intrinsics[:, :, 1, 2]),
        bn(intrinsics[:, :, 0, 0]), bn(intrinsics[:, :, 1, 1]),
        RnB,
        tnB,
        RtB,
        ttB,
        bt(target_intrinsics[:, :, 0, 0]), bt(target_intrinsics[:, :, 1, 1]),
        bt(target_intrinsics[:, :, 0, 2]), bt(target_intrinsics[:, :, 1, 2]),
    ], axis=-1).astype(f32).reshape(B * T * N, 32)

    depths_fl = depths.reshape(B * N, H, W)
    pix, zc = _project(depths_fl, params)
    pix2 = pix.reshape(NR * P)
    zc2 = zc.reshape(NR * P)
    colsf = images.reshape(B * N * C * H * W)
    return pix2, zc2, colsf


def kernel(images, depths, extrinsics, intrinsics,
           target_extrinsics, target_intrinsics):
    pix2, zc2, colsf = _frontend(images, depths, extrinsics, intrinsics,
                                 target_extrinsics, target_intrinsics)
    outr, outg, outb = (o.reshape(NR, ZP)[:, :HWD]
                        for o in _sc_rasterize(pix2, zc2, colsf))
    img = jnp.stack([outr, outg, outb], axis=-1)
    return img.reshape(B, T, H, W, C)
